# Initial kernel scaffold; baseline (speedup 1.0000x reference)
import jax
import jax.numpy as jnp
from jax.experimental import pallas as pl


def kernel(z, means, logvars, w):
    raise NotImplementedError("write your pallas kernel here")


# speedup vs baseline: 1.7511x; 1.7511x over previous
import math

import jax
import jax.numpy as jnp
from jax.experimental import pallas as pl
from jax.experimental.pallas import tpu as pltpu

B, L, K = 16384, 64, 64
PACK = 2
W2 = L * PACK
B2 = B // PACK
TB = 512
SHIFT = 40.0
NEG_HALF_LOG_2PI = -0.5 * math.log(2.0 * math.pi)


def _mog_body(z_ref, mu_ref, lv_ref, w_ref, out_ref):
    mu = mu_ref[:]
    lv = lv_ref[:]
    wv = w_ref[:]

    wmax = jnp.max(wv, axis=0, keepdims=True)
    lw = wv - (wmax + jnp.log(jnp.sum(jnp.exp(wv - wmax), axis=0, keepdims=True)))

    p = jnp.exp(-lv)
    c = p * mu
    d = -0.5 * p
    a = (NEG_HALF_LOG_2PI + SHIFT) - 0.5 * lv - 0.5 * p * mu * mu + lw

    z = z_ref[:]
    s = jnp.zeros_like(z)
    for k in range(K):
        v = a[k : k + 1, :] + z * (c[k : k + 1, :] + d[k : k + 1, :] * z)
        s = s + jnp.exp(v)
    out_ref[:] = jnp.log(s) - SHIFT


def kernel(z, means, logvars, w):
    z2 = z.reshape(B2, W2)
    mu2 = jnp.tile(means, (1, PACK))
    lv2 = jnp.tile(logvars, (1, PACK))
    w2 = jnp.broadcast_to(w.reshape(K, 1), (K, W2))

    grid = (B2 // TB,)
    out2 = pl.pallas_call(
        _mog_body,
        grid=grid,
        in_specs=[
            pl.BlockSpec((TB, W2), lambda i: (i, 0)),
            pl.BlockSpec((K, W2), lambda i: (0, 0)),
            pl.BlockSpec((K, W2), lambda i: (0, 0)),
            pl.BlockSpec((K, W2), lambda i: (0, 0)),
        ],
        out_specs=pl.BlockSpec((TB, W2), lambda i: (i, 0)),
        out_shape=jax.ShapeDtypeStruct((B2, W2), jnp.float32),
    )(z2, mu2, lv2, w2)
    return out2.reshape(B, L)


# speedup vs baseline: 1.9145x; 1.0933x over previous
import math

import jax
import jax.numpy as jnp
from jax.experimental import pallas as pl
from jax.experimental.pallas import tpu as pltpu

B, L, K = 16384, 64, 64
PACK = 2
W2 = L * PACK
B2 = B // PACK
TB = 512
SHIFT = 40.0
NEG_HALF_LOG_2PI = -0.5 * math.log(2.0 * math.pi)
LOG2E = math.log2(math.e)
LN2 = math.log(2.0)


def _mog_body(z_ref, mu_ref, lv_ref, w_ref, out_ref):
    mu = mu_ref[:]
    lv = lv_ref[:]
    wv = w_ref[:]

    wmax = jnp.max(wv, axis=0, keepdims=True)
    lw = wv - (wmax + jnp.log(jnp.sum(jnp.exp(wv - wmax), axis=0, keepdims=True)))

    p = jnp.exp(-lv)
    c = LOG2E * (p * mu)
    d = LOG2E * (-0.5 * p)
    a = LOG2E * ((NEG_HALF_LOG_2PI + SHIFT) - 0.5 * lv - 0.5 * p * mu * mu + lw)

    z = z_ref[:]
    s = jnp.zeros_like(z)
    for k in range(K):
        v = a[k : k + 1, :] + z * (c[k : k + 1, :] + d[k : k + 1, :] * z)
        s = s + jnp.exp2(v)
    out_ref[:] = LN2 * jnp.log2(s) - SHIFT


def kernel(z, means, logvars, w):
    z2 = z.reshape(B2, W2)
    mu2 = jnp.tile(means, (1, PACK))
    lv2 = jnp.tile(logvars, (1, PACK))
    w2 = jnp.broadcast_to(w.reshape(K, 1), (K, W2))

    grid = (B2 // TB,)
    out2 = pl.pallas_call(
        _mog_body,
        grid=grid,
        in_specs=[
            pl.BlockSpec((TB, W2), lambda i: (i, 0)),
            pl.BlockSpec((K, W2), lambda i: (0, 0)),
            pl.BlockSpec((K, W2), lambda i: (0, 0)),
            pl.BlockSpec((K, W2), lambda i: (0, 0)),
        ],
        out_specs=pl.BlockSpec((TB, W2), lambda i: (i, 0)),
        out_shape=jax.ShapeDtypeStruct((B2, W2), jnp.float32),
    )(z2, mu2, lv2, w2)
    return out2.reshape(B, L)
